# baseline (speedup 1.0000x reference)
import functools

import jax
import jax.numpy as jnp
from jax import lax
from jax.experimental import pallas as pl
from jax.experimental.pallas import tpu as pltpu
from jax.experimental.pallas import tpu_sc as plsc

RESO = 128
NPTS = 2_000_000
BLK = 2000
NBLK = NPTS // BLK
NWORKERS = 32
CHUNKS = BLK // 16

_F32 = jnp.float32
_I32 = jnp.int32


def _rsqrt(s):
    i = lax.bitcast_convert_type(s, _I32)
    i = jnp.int32(0x5F3759DF) - lax.shift_right_arithmetic(i, 1)
    y = lax.bitcast_convert_type(i, _F32)
    for _ in range(3):
        y = y * (1.5 - 0.5 * s * y * y)
    return y


def _body(rpts_h, ridx_h, spts_h, sidx_h, xl_h, yl_h, zl_h, off_h,
          sdf_out_h, nrm_out_h,
          xl_v, yl_v, zl_v, off_v, idx_v, pts_v, sdf_v, nrm_v):
    w = lax.axis_index("s") * 2 + lax.axis_index("c")
    pltpu.sync_copy(xl_h, xl_v)
    pltpu.sync_copy(yl_h, yl_v)
    pltpu.sync_copy(zl_h, zl_v)
    pltpu.sync_copy(off_h, off_v)

    iota3 = lax.broadcasted_iota(_I32, (16,), 0) * 3
    d = off_v[0]
    e = off_v[1]
    f = off_v[2]
    g = off_v[3]

    nblk_w = (NBLK - 1 - w) // NWORKERS + 1

    def gather_chunk(i):
        s = i * 16
        idx = idx_v[pl.ds(s, 16)]
        iz = lax.bitwise_and(idx, 127)
        iy = lax.bitwise_and(lax.shift_right_logical(idx, 7), 127)
        ix = lax.bitwise_and(lax.shift_right_logical(idx, 14), 127)
        a = plsc.load_gather(xl_v, [ix])
        b = plsc.load_gather(yl_v, [iy])
        c = plsc.load_gather(zl_v, [iz])
        p3 = iota3 + s * 3
        px = plsc.load_gather(pts_v, [p3]) + ix.astype(_F32)
        py = plsc.load_gather(pts_v, [p3 + 1]) + iy.astype(_F32)
        pz = plsc.load_gather(pts_v, [p3 + 2]) + iz.astype(_F32)
        return a, b, c, px, py, pz, p3

    def sdf_block(t, carry):
        base = (w + t * NWORKERS) * BLK
        pltpu.sync_copy(sidx_h.at[pl.ds(base, BLK)], idx_v)
        pltpu.sync_copy(spts_h.at[pl.ds(3 * base, 3 * BLK)], pts_v)

        def chunk(i, c2):
            a, b, c, px, py, pz, _ = gather_chunk(i)
            val = px * (a * px + d) + py * (b * py + e) + pz * (c * pz + f) + g
            sdf_v[pl.ds(i * 16, 16)] = val
            return c2

        lax.fori_loop(0, CHUNKS, chunk, 0)
        pltpu.sync_copy(sdf_v, sdf_out_h.at[pl.ds(base, BLK)])
        return carry

    def nrm_block(t, carry):
        base = (w + t * NWORKERS) * BLK
        pltpu.sync_copy(ridx_h.at[pl.ds(base, BLK)], idx_v)
        pltpu.sync_copy(rpts_h.at[pl.ds(3 * base, 3 * BLK)], pts_v)

        def chunk(i, c2):
            a, b, c, px, py, pz, p3 = gather_chunk(i)
            gx = 2.0 * a * px + d
            gy = 2.0 * b * py + e
            gz = 2.0 * c * pz + f
            s2 = gx * gx + gy * gy + gz * gz
            norm = s2 * _rsqrt(s2)
            inv = 1.0 / (norm + 1e-8)
            plsc.store_scatter(nrm_v, [p3], gx * inv)
            plsc.store_scatter(nrm_v, [p3 + 1], gy * inv)
            plsc.store_scatter(nrm_v, [p3 + 2], gz * inv)
            return c2

        lax.fori_loop(0, CHUNKS, chunk, 0)
        pltpu.sync_copy(nrm_v, nrm_out_h.at[pl.ds(3 * base, 3 * BLK)])
        return carry

    lax.fori_loop(0, nblk_w, sdf_block, 0)
    lax.fori_loop(0, nblk_w, nrm_block, 0)


_sc_call = functools.partial(
    pl.kernel,
    out_type=[
        jax.ShapeDtypeStruct((NPTS,), _F32),
        jax.ShapeDtypeStruct((3 * NPTS,), _F32),
    ],
    mesh=plsc.VectorSubcoreMesh(core_axis_name="c", subcore_axis_name="s"),
    compiler_params=pltpu.CompilerParams(needs_layout_passes=False),
    scratch_types=[
        pltpu.VMEM((RESO,), _F32),
        pltpu.VMEM((RESO,), _F32),
        pltpu.VMEM((RESO,), _F32),
        pltpu.VMEM((4, 16), _F32),
        pltpu.VMEM((BLK,), _I32),
        pltpu.VMEM((3 * BLK,), _F32),
        pltpu.VMEM((BLK,), _F32),
        pltpu.VMEM((3 * BLK,), _F32),
    ],
)(_body)


def kernel(renderPointList, renderIndexList, sdfPointList, sdfIndexList,
           xLayer, yLayer, zLayer, offset):
    off16 = jnp.broadcast_to(offset[:, None], (4, 16))
    sdf, nrm = _sc_call(
        renderPointList.reshape(-1), renderIndexList,
        sdfPointList.reshape(-1), sdfIndexList,
        xLayer, yLayer, zLayer, off16)
    return (sdf, nrm.reshape(NPTS, 3))
